# baseline (speedup 1.0000x reference)
import functools

import jax
from jax import lax
import jax.numpy as jnp
from jax.experimental import pallas as pl
from jax.experimental.pallas import tpu as pltpu
from jax.experimental.pallas import tpu_sc as plsc

_B, _S, _D, _E, _K, _F, _FS = 64, 1, 2048, 64, 8, 512, 512
_T = _B * _S
_FB = 512
_NF = _F // _FB
_NW = 32
_TPW = _T // _NW
_NSL = _E // 16


def _dot_t(a, b, precision=None):
    return jax.lax.dot_general(
        a, b, (((1,), (1,)), ((), ())),
        preferred_element_type=jnp.float32, precision=precision)


def _dot_t_bf16(a, b):
    return jax.lax.dot_general(
        a.astype(jnp.bfloat16), b.astype(jnp.bfloat16),
        (((1,), (1,)), ((), ())), preferred_element_type=jnp.float32)


def _logits_kernel(x_ref, rw_ref, sgw_ref, logits_ref, sg_ref):
    x = x_ref[...]
    logits_ref[...] = _dot_t(x, rw_ref[...])
    sg = _dot_t_bf16(x, sgw_ref[...])
    sg_ref[...] = sg * jax.nn.sigmoid(sg)


def _shared_kernel(x_ref, sg_ref, suw_ref, sdw_ref, segw_ref, out_ref):
    x = x_ref[...]
    su = _dot_t_bf16(x, suw_ref[...])
    sh = sg_ref[...] * su
    sd = _dot_t_bf16(sh, sdw_ref[...])
    tok_gate = jax.nn.sigmoid(_dot_t(x, segw_ref[...]))
    out_ref[...] = tok_gate * sd


def _route_sc_kernel(logits_hbm, w_hbm, rows_v, out_v):
    wid = lax.axis_index("s") * 2 + lax.axis_index("c")
    base = wid * _TPW
    pltpu.sync_copy(logits_hbm.at[pl.ds(base, _TPW)], rows_v)
    lanes = lax.broadcasted_iota(jnp.int32, (16,), 0)
    for ti in range(_TPW):
        r = [rows_v[ti, pl.ds(16 * j, 16)] for j in range(_NSL)]
        row_max = jnp.max(jnp.maximum(jnp.maximum(r[0], r[1]),
                                      jnp.maximum(r[2], r[3])), axis=0)
        p = [jnp.exp(r[j] - row_max) for j in range(_NSL)]
        acc = [jnp.zeros((16,), jnp.float32) for _ in range(_NSL)]
        rem = list(r)
        for _ in range(_K):
            m = jnp.max(jnp.maximum(jnp.maximum(rem[0], rem[1]),
                                    jnp.maximum(rem[2], rem[3])), axis=0)
            first = jnp.int32(_E)
            for j in range(_NSL):
                cand = jnp.where(rem[j] == m, lanes + 16 * j, _E)
                first = jnp.minimum(first, jnp.min(cand, axis=0))
            for j in range(_NSL):
                pick = (lanes + 16 * j) == first
                acc[j] = jnp.where(pick, p[j], acc[j])
                rem[j] = jnp.where(pick, -jnp.inf, rem[j])
        s = jnp.float32(0.0)
        for j in range(_NSL):
            s = s + jnp.sum(acc[j], axis=0)
        for j in range(_NSL):
            out_v[ti, pl.ds(16 * j, 16)] = acc[j] / s
    pltpu.sync_copy(out_v, w_hbm.at[pl.ds(base, _TPW)])


_route_sc = functools.partial(
    pl.kernel,
    mesh=plsc.VectorSubcoreMesh(core_axis_name="c", subcore_axis_name="s"),
    out_type=jax.ShapeDtypeStruct((_T, _E), jnp.float32),
    scratch_types=[
        pltpu.VMEM((_TPW, _E), jnp.float32),
        pltpu.VMEM((_TPW, _E), jnp.float32),
    ],
    compiler_params=pltpu.CompilerParams(needs_layout_passes=False),
)(_route_sc_kernel)


def _moe_kernel(x_ref, w_ref, shared_ref, gw_ref, uw_ref, dw_ref, out_ref):
    e = pl.program_id(0)
    fi = pl.program_id(1)
    x = x_ref[...]

    @pl.when(jnp.logical_and(e == 0, fi == 0))
    def _prologue():
        out_ref[...] = shared_ref[...]

    w_e = jnp.sum(
        jnp.where(jax.lax.broadcasted_iota(jnp.int32, (_T, _E), 1) == e,
                  w_ref[...], 0.0),
        axis=1, keepdims=True)
    g = _dot_t_bf16(x, gw_ref[0])
    u = _dot_t_bf16(x, uw_ref[0])
    h = (g * jax.nn.sigmoid(g)) * u * w_e
    contrib = _dot_t_bf16(h, dw_ref[0])
    out_ref[...] += contrib


@jax.jit
def kernel(hidden_states, router_w, expert_gate_w, expert_up_w, expert_down_w,
           shared_gate_w, shared_up_w, shared_down_w, shared_expert_gate_w):
    x = hidden_states.reshape(_T, _D)
    logits, sg_act = pl.pallas_call(
        _logits_kernel,
        out_shape=[
            jax.ShapeDtypeStruct((_T, _E), jnp.float32),
            jax.ShapeDtypeStruct((_T, _FS), jnp.float32),
        ],
    )(x, router_w, shared_gate_w)
    w = _route_sc(logits)
    shared_out = pl.pallas_call(
        _shared_kernel,
        out_shape=jax.ShapeDtypeStruct((_T, _D), jnp.float32),
    )(x, sg_act, shared_up_w, shared_down_w, shared_expert_gate_w)
    out = pl.pallas_call(
        _moe_kernel,
        grid=(_E, _NF),
        in_specs=[
            pl.BlockSpec((_T, _D), lambda e, f: (0, 0)),
            pl.BlockSpec((_T, _E), lambda e, f: (0, 0)),
            pl.BlockSpec((_T, _D), lambda e, f: (0, 0)),
            pl.BlockSpec((1, _FB, _D), lambda e, f: (e, f, 0)),
            pl.BlockSpec((1, _FB, _D), lambda e, f: (e, f, 0)),
            pl.BlockSpec((1, _D, _FB), lambda e, f: (e, 0, f)),
        ],
        out_specs=pl.BlockSpec((_T, _D), lambda e, f: (0, 0)),
        out_shape=jax.ShapeDtypeStruct((_T, _D), jnp.float32),
        compiler_params=pltpu.CompilerParams(
            dimension_semantics=("arbitrary", "arbitrary")),
    )(x, w, shared_out, expert_gate_w, expert_up_w, expert_down_w)
    return out.reshape(_B, _S, _D), logits


# speedup vs baseline: 1.0028x; 1.0028x over previous
import functools

import jax
from jax import lax
import jax.numpy as jnp
from jax.experimental import pallas as pl
from jax.experimental.pallas import tpu as pltpu
from jax.experimental.pallas import tpu_sc as plsc

_B, _S, _D, _E, _K, _F, _FS = 64, 1, 2048, 64, 8, 512, 512
_T = _B * _S
_FB = 512
_NF = _F // _FB
_NW = 32
_TPW = _T // _NW
_NSL = _E // 16


def _dot_t(a, b, precision=None):
    return jax.lax.dot_general(
        a, b, (((1,), (1,)), ((), ())),
        preferred_element_type=jnp.float32, precision=precision)


def _dot_t_bf16(a, b):
    return jax.lax.dot_general(
        a.astype(jnp.bfloat16), b.astype(jnp.bfloat16),
        (((1,), (1,)), ((), ())), preferred_element_type=jnp.float32)


def _logits_kernel(x_ref, rw_ref, logits_ref):
    logits_ref[...] = _dot_t(x_ref[...], rw_ref[...])


def _shared_kernel(x_ref, sgw_ref, suw_ref, sdw_ref, segw_ref, out_ref):
    fi = pl.program_id(0)
    x = x_ref[...]
    sg = _dot_t_bf16(x, sgw_ref[...])
    su = _dot_t_bf16(x, suw_ref[...])
    sh = (sg * jax.nn.sigmoid(sg)) * su
    tok_gate = jax.nn.sigmoid(_dot_t(x, segw_ref[...]))
    part = tok_gate * _dot_t_bf16(sh, sdw_ref[...])

    @pl.when(fi == 0)
    def _init():
        out_ref[...] = part

    @pl.when(fi != 0)
    def _acc():
        out_ref[...] += part


def _route_sc_kernel(logits_hbm, w_hbm, rows_v, out_v):
    wid = lax.axis_index("s") * 2 + lax.axis_index("c")
    base = wid * _TPW
    pltpu.sync_copy(logits_hbm.at[pl.ds(base, _TPW)], rows_v)
    lanes = lax.broadcasted_iota(jnp.int32, (16,), 0)
    for ti in range(_TPW):
        r = [rows_v[ti, pl.ds(16 * j, 16)] for j in range(_NSL)]
        row_max = jnp.max(jnp.maximum(jnp.maximum(r[0], r[1]),
                                      jnp.maximum(r[2], r[3])), axis=0)
        p = [jnp.exp(r[j] - row_max) for j in range(_NSL)]
        acc = [jnp.zeros((16,), jnp.float32) for _ in range(_NSL)]
        rem = list(r)
        for _ in range(_K):
            m = jnp.max(jnp.maximum(jnp.maximum(rem[0], rem[1]),
                                    jnp.maximum(rem[2], rem[3])), axis=0)
            first = jnp.int32(_E)
            for j in range(_NSL):
                cand = jnp.where(rem[j] == m, lanes + 16 * j, _E)
                first = jnp.minimum(first, jnp.min(cand, axis=0))
            for j in range(_NSL):
                pick = (lanes + 16 * j) == first
                acc[j] = jnp.where(pick, p[j], acc[j])
                rem[j] = jnp.where(pick, -jnp.inf, rem[j])
        s = jnp.float32(0.0)
        for j in range(_NSL):
            s = s + jnp.sum(acc[j], axis=0)
        for j in range(_NSL):
            out_v[ti, pl.ds(16 * j, 16)] = acc[j] / s
    pltpu.sync_copy(out_v, w_hbm.at[pl.ds(base, _TPW)])


_route_sc = functools.partial(
    pl.kernel,
    mesh=plsc.VectorSubcoreMesh(core_axis_name="c", subcore_axis_name="s"),
    out_type=jax.ShapeDtypeStruct((_T, _E), jnp.float32),
    scratch_types=[
        pltpu.VMEM((_TPW, _E), jnp.float32),
        pltpu.VMEM((_TPW, _E), jnp.float32),
    ],
    compiler_params=pltpu.CompilerParams(needs_layout_passes=False),
)(_route_sc_kernel)


def _moe_kernel(x_ref, w_ref, shared_ref, gw_ref, uw_ref, dw_ref, out_ref):
    e = pl.program_id(0)
    fi = pl.program_id(1)
    x = x_ref[...]

    @pl.when(jnp.logical_and(e == 0, fi == 0))
    def _prologue():
        out_ref[...] = shared_ref[...]

    w_e = jnp.sum(
        jnp.where(jax.lax.broadcasted_iota(jnp.int32, (_T, _E), 1) == e,
                  w_ref[...], 0.0),
        axis=1, keepdims=True)
    g = _dot_t_bf16(x, gw_ref[0])
    u = _dot_t_bf16(x, uw_ref[0])
    h = (g * jax.nn.sigmoid(g)) * u * w_e
    contrib = _dot_t_bf16(h, dw_ref[0])
    out_ref[...] += contrib


@jax.jit
def kernel(hidden_states, router_w, expert_gate_w, expert_up_w, expert_down_w,
           shared_gate_w, shared_up_w, shared_down_w, shared_expert_gate_w):
    x = hidden_states.reshape(_T, _D)
    logits = pl.pallas_call(
        _logits_kernel,
        out_shape=jax.ShapeDtypeStruct((_T, _E), jnp.float32),
    )(x, router_w)
    w = _route_sc(logits)
    shared_out = pl.pallas_call(
        _shared_kernel,
        grid=(2,),
        in_specs=[
            pl.BlockSpec((_T, _D), lambda f: (0, 0)),
            pl.BlockSpec((_FS // 2, _D), lambda f: (f, 0)),
            pl.BlockSpec((_FS // 2, _D), lambda f: (f, 0)),
            pl.BlockSpec((_D, _FS // 2), lambda f: (0, f)),
            pl.BlockSpec((1, _D), lambda f: (0, 0)),
        ],
        out_specs=pl.BlockSpec((_T, _D), lambda f: (0, 0)),
        out_shape=jax.ShapeDtypeStruct((_T, _D), jnp.float32),
        compiler_params=pltpu.CompilerParams(
            dimension_semantics=("arbitrary",)),
    )(x, shared_gate_w, shared_up_w, shared_down_w, shared_expert_gate_w)
    out = pl.pallas_call(
        _moe_kernel,
        grid=(_E, _NF),
        in_specs=[
            pl.BlockSpec((_T, _D), lambda e, f: (0, 0)),
            pl.BlockSpec((_T, _E), lambda e, f: (0, 0)),
            pl.BlockSpec((_T, _D), lambda e, f: (0, 0)),
            pl.BlockSpec((1, _FB, _D), lambda e, f: (e, f, 0)),
            pl.BlockSpec((1, _FB, _D), lambda e, f: (e, f, 0)),
            pl.BlockSpec((1, _D, _FB), lambda e, f: (e, 0, f)),
        ],
        out_specs=pl.BlockSpec((_T, _D), lambda e, f: (0, 0)),
        out_shape=jax.ShapeDtypeStruct((_T, _D), jnp.float32),
        compiler_params=pltpu.CompilerParams(
            dimension_semantics=("arbitrary", "arbitrary")),
    )(x, w, shared_out, expert_gate_w, expert_up_w, expert_down_w)
    return out.reshape(_B, _S, _D), logits
